# baseline (speedup 1.0000x reference)
import functools

import jax
import jax.numpy as jnp
from jax import lax
from jax.experimental import pallas as pl
from jax.experimental.pallas import tpu as pltpu

N_USERS = 5000
N_ITEMS = 50000
N_TOTAL = N_USERS + N_ITEMS
HIDDEN = 64
NNZ = 640000
BATCH = 4096
DECAY = 1e-4
CL_REG = 0.01
TEMP = 0.2


_R = 200
_CHUNKS = ((0, 1280), (1280, 1280), (2560, 1280), (3840, 1160))


def _normalize_rows(x):
    return x / (jnp.sqrt(jnp.sum(x * x, axis=1, keepdims=True)) + 1e-8)


def _pass1_body(s_ref, ui_ref, u_ref, x_ref, y_ref, d_ref):
    uni = _normalize_rows(ui_ref[...])
    acc = jnp.zeros((_R, HIDDEN), jnp.float32)
    accd = jnp.zeros((_R, 1), jnp.float32)
    for c0, w in _CHUNKS:
        unj = _normalize_rows(u_ref[c0:c0 + w, :])
        sim = jax.nn.relu(jax.lax.dot_general(
            uni, unj, (((1,), (1,)), ((), ())),
            preferred_element_type=jnp.float32))
        m = 0.5 * s_ref[:, c0:c0 + w] + 0.5 * sim
        acc += jnp.dot(m.astype(jnp.bfloat16),
                       x_ref[c0:c0 + w, :].astype(jnp.bfloat16),
                       preferred_element_type=jnp.float32)
        accd += jnp.sum(m, axis=1, keepdims=True)
    d = accd + 1e-8
    y_ref[...] = acc / d
    d_ref[...] = d


def _pass2_body(s_ref, ui_ref, u_ref, x_ref, din_ref, y_ref):
    uni = _normalize_rows(ui_ref[...])
    k2 = x_ref.shape[1]
    acc = jnp.zeros((_R, k2), jnp.float32)
    for c0, w in _CHUNKS:
        unj = _normalize_rows(u_ref[c0:c0 + w, :])
        sim = jax.nn.relu(jax.lax.dot_general(
            uni, unj, (((1,), (1,)), ((), ())),
            preferred_element_type=jnp.float32))
        m = 0.5 * s_ref[:, c0:c0 + w] + 0.5 * sim
        acc += jnp.dot(m.astype(jnp.bfloat16),
                       x_ref[c0:c0 + w, :].astype(jnp.bfloat16),
                       preferred_element_type=jnp.float32)
    y_ref[...] = acc / din_ref[...]


def _graph_pass1(S, u):
    ni = N_USERS // _R
    return pl.pallas_call(
        _pass1_body,
        grid=(ni,),
        in_specs=[
            pl.BlockSpec((_R, N_USERS), lambda i: (i, 0)),
            pl.BlockSpec((_R, HIDDEN), lambda i: (i, 0)),
            pl.BlockSpec((N_USERS, HIDDEN), lambda i: (0, 0)),
            pl.BlockSpec((N_USERS, HIDDEN), lambda i: (0, 0)),
        ],
        out_specs=[
            pl.BlockSpec((_R, HIDDEN), lambda i: (i, 0)),
            pl.BlockSpec((_R, 1), lambda i: (i, 0)),
        ],
        out_shape=[
            jax.ShapeDtypeStruct((N_USERS, HIDDEN), jnp.float32),
            jax.ShapeDtypeStruct((N_USERS, 1), jnp.float32),
        ],
    )(S, u, u, u)


def _graph_pass2(S, u, x2, d):
    ni = N_USERS // _R
    k2 = x2.shape[1]
    return pl.pallas_call(
        _pass2_body,
        grid=(ni,),
        in_specs=[
            pl.BlockSpec((_R, N_USERS), lambda i: (i, 0)),
            pl.BlockSpec((_R, HIDDEN), lambda i: (i, 0)),
            pl.BlockSpec((N_USERS, HIDDEN), lambda i: (0, 0)),
            pl.BlockSpec((N_USERS, k2), lambda i: (0, 0)),
            pl.BlockSpec((_R, 1), lambda i: (i, 0)),
        ],
        out_specs=pl.BlockSpec((_R, k2), lambda i: (i, 0)),
        out_shape=jax.ShapeDtypeStruct((N_USERS, k2), jnp.float32),
    )(S, u, u, x2, d)


_B = 512


def _loss_body(gi_ref, gj_ref, fp_ref, fn_ref, ip_ref, in_ref,
               out_ref, acct_ref, accs_ref):
    i = pl.program_id(0)
    j = pl.program_id(1)
    ni = pl.num_programs(0)
    nj = pl.num_programs(1)

    @pl.when(j == 0)
    def _init():
        acct_ref[...] = jnp.zeros_like(acct_ref)

    gi = gi_ref[...]
    z1n = _normalize_rows(gi[:, HIDDEN:2 * HIDDEN])
    z2n = _normalize_rows(gj_ref[:, 0:HIDDEN])
    logits = jax.lax.dot_general(
        z1n, z2n, (((1,), (1,)), ((), ())), preferred_element_type=jnp.float32)
    acct_ref[...] += jnp.sum(jnp.exp(logits / TEMP), axis=1, keepdims=True)

    @pl.when(j == nj - 1)
    def _fini():
        z2ni = _normalize_rows(gi[:, 0:HIDDEN])
        posv = jnp.exp(jnp.sum(z1n * z2ni, axis=1, keepdims=True) / TEMP)
        tot = acct_ref[...]
        cl_part = -jnp.sum(jnp.log(posv / (tot + 1e-8) + 1e-8)) / BATCH

        uo = gi[:, 2 * HIDDEN:3 * HIDDEN]
        ps = jnp.sum(uo * fp_ref[...], axis=1)
        ns = jnp.sum(uo * fn_ref[...], axis=1)
        x = ns - ps
        softplus = jnp.maximum(x, 0.0) + jnp.log1p(jnp.exp(-jnp.abs(x)))
        bpr_part = jnp.sum(softplus) / BATCH

        ue = gi[:, 3 * HIDDEN:4 * HIDDEN]
        reg_part = 0.5 * (jnp.sum(ue ** 2) + jnp.sum(ip_ref[...] ** 2)
                          + jnp.sum(in_ref[...] ** 2)) / BATCH

        partial = bpr_part + DECAY * reg_part + CL_REG * cl_part

        @pl.when(i == 0)
        def _zero():
            accs_ref[0] = 0.0

        accs_ref[0] += partial

        @pl.when(i == ni - 1)
        def _emit():
            out_ref[0, 0] = accs_ref[0]


def _loss_kernel(G, FPN, IPN):
    ni = nj = BATCH // _B
    nb = ni
    spec_gi = pl.BlockSpec((_B, 4 * HIDDEN), lambda i, j: (i, 0))
    spec_gj = pl.BlockSpec((_B, 4 * HIDDEN), lambda i, j: (j, 0))
    spec_p = pl.BlockSpec((_B, HIDDEN), lambda i, j: (i, 0))
    spec_n = pl.BlockSpec((_B, HIDDEN), lambda i, j: (i + nb, 0))
    out = pl.pallas_call(
        _loss_body,
        grid=(ni, nj),
        in_specs=[spec_gi, spec_gj, spec_p, spec_n, spec_p, spec_n],
        out_specs=pl.BlockSpec(memory_space=pltpu.SMEM),
        out_shape=jax.ShapeDtypeStruct((1, 1), jnp.float32),
        scratch_shapes=[
            pltpu.VMEM((_B, 1), jnp.float32),
            pltpu.SMEM((1,), jnp.float32),
        ],
    )(G, G, FPN, FPN, IPN, IPN)
    return out[0, 0]


from jax.experimental.pallas import tpu_sc as plsc

_NT_PAD = 55040
_RP = _NT_PAD // 16
_RB = 20
_EB = 256
_CH = 40960
_NB = _CH // _EB
_NNZ_PAD = 16 * _CH
_HALF = HIDDEN // 2


def _spmm_sc_kernel(eA, eB, rows, cols, vals):
    mesh = plsc.VectorSubcoreMesh(core_axis_name="c", subcore_axis_name="s")
    fdt = jnp.float32
    idt = jnp.int32
    out_t = [jax.ShapeDtypeStruct((_NT_PAD, _HALF), fdt) for _ in range(4)]

    @functools.partial(
        pl.kernel,
        out_type=out_t,
        mesh=mesh,
        compiler_params=pltpu.CompilerParams(use_tc_tiling_on_sc=False),
        scratch_types=[
            pltpu.VMEM_SHARED((_NT_PAD, _HALF), fdt),
            pltpu.VMEM((_EB,), idt), pltpu.VMEM((_EB,), idt),
            pltpu.VMEM((_EB,), idt), pltpu.VMEM((_EB,), idt),
            pltpu.VMEM((_EB,), fdt), pltpu.VMEM((_EB,), fdt),
            pltpu.VMEM((_EB,), idt), pltpu.VMEM((_EB,), idt),
            pltpu.VMEM((_EB, _HALF), fdt), pltpu.VMEM((_EB, _HALF), fdt),
            pltpu.VMEM((_RB, _HALF), fdt),
            pltpu.VMEM((_RB, _HALF), fdt),
            pltpu.SemaphoreType.DMA, pltpu.SemaphoreType.DMA,
            pltpu.SemaphoreType.DMA, pltpu.SemaphoreType.DMA,
            pltpu.SemaphoreType.DMA, pltpu.SemaphoreType.DMA,
            pltpu.SemaphoreType.DMA, pltpu.SemaphoreType.DMA,
            pltpu.SemaphoreType.DMA, pltpu.SemaphoreType.DMA,
        ],
    )
    def spmm(eA_h, eB_h, rows_h, cols_h, vals_h,
             outA_h, outB_h, e1A_h, e1B_h,
             acc, rowi0, rowi1, coli0, coli1, valb0, valb1, srow0, srow1,
             gbuf0, gbuf1, sbuf, tbuf,
             isem0, isem1, gsem0, gsem1, ssem0, ssem1,
             lsem0, lsem1, lsem2, lsem3):
        c = lax.axis_index("c")
        t = lax.axis_index("s")
        zeros16 = jnp.zeros((16,), fdt)

        def zero_slab(r, _):
            sbuf[r, 0:16] = zeros16
            sbuf[r, 16:32] = zeros16
            return 0

        bufs = ((rowi0, coli0, valb0, srow0, gbuf0, isem0, gsem0, ssem0),
                (rowi1, coli1, valb1, srow1, gbuf1, isem1, gsem1, ssem1))

        def fetch(b, ri, ci, vb, sem):
            base = t * _CH + b * _EB
            pltpu.async_copy(rows_h.at[pl.ds(base, _EB)], ri, sem)
            pltpu.async_copy(cols_h.at[pl.ds(base, _EB)], ci, sem)
            pltpu.async_copy(vals_h.at[pl.ds(base, _EB)], vb, sem)

        def fetch_wait(ri, ci, vb, sem):
            pltpu.make_async_copy(rows_h.at[pl.ds(0, _EB)], ri, sem).wait()
            pltpu.make_async_copy(cols_h.at[pl.ds(0, _EB)], ci, sem).wait()
            pltpu.make_async_copy(vals_h.at[pl.ds(0, _EB)], vb, sem).wait()

        def scale_copy(gb, vb, ri, sr):
            def scale(g, _):
                v16 = vb[pl.ds(g * 16, 16)]
                sr[pl.ds(g * 16, 16)] = ri[pl.ds(g * 16, 16)]
                for k in range(16):
                    vk = lax.gather(
                        v16, jnp.full((16, 1), k, jnp.int32),
                        lax.GatherDimensionNumbers(
                            offset_dims=(), collapsed_slice_dims=(0,),
                            start_index_map=(0,)),
                        slice_sizes=(1,),
                        mode=lax.GatherScatterMode.PROMISE_IN_BOUNDS)
                    e = g * 16 + k
                    gb[e, 0:16] *= vk
                    gb[e, 16:32] *= vk
                return 0

            lax.fori_loop(0, _EB // 16, scale, 0)

        def hop(src_h):
            def step(b, p):
                ri, ci, vb, sr, gb, isem, gsem, ssem = bufs[p]
                ri2, ci2, vb2, sr2, gb2, isem2, gsem2, ssem2 = bufs[1 - p]
                pltpu.make_async_copy(src_h.at[ci], gb, gsem).wait()

                @pl.when(b + 1 < _NB)
                def _issue_next():
                    fetch_wait(ri2, ci2, vb2, isem2)

                    @pl.when(b >= 1)
                    def _drain_prev_scatter():
                        pltpu.make_async_copy(gb2, acc.at[sr2], ssem2).wait()

                    pltpu.async_copy(src_h.at[ci2], gb2, gsem2)

                scale_copy(gb, vb, ri, sr)
                pltpu.async_copy(gb, acc.at[sr], ssem, add=True)

                @pl.when(b + 2 < _NB)
                def _prefetch():
                    fetch(b + 2, ri, ci, vb, isem)

            def pair(b2, _):
                step(2 * b2, 0)
                step(2 * b2 + 1, 1)
                return 0

            fetch(0, rowi0, coli0, valb0, isem0)
            fetch(1, rowi1, coli1, valb1, isem1)
            fetch_wait(rowi0, coli0, valb0, isem0)
            pltpu.async_copy(src_h.at[coli0], gbuf0, gsem0)
            lax.fori_loop(0, _NB // 2, pair, 0)
            pltpu.make_async_copy(gbuf0, acc.at[srow0], ssem0).wait()
            pltpu.make_async_copy(gbuf1, acc.at[srow1], ssem1).wait()

        _NS = _RP // _RB

        def run(e_h, e1_h, out_h):
            slab0 = t * _RP

            lax.fori_loop(0, _RB, zero_slab, 0)

            def zfire(s, _):
                pltpu.async_copy(sbuf, acc.at[pl.ds(slab0 + s * _RB, _RB)],
                                 ssem0)
                return 0

            def zdrain(s, _):
                pltpu.make_async_copy(sbuf, acc.at[pl.ds(0, _RB)],
                                      ssem0).wait()
                return 0

            lax.fori_loop(0, _NS, zfire, 0)
            lax.fori_loop(0, _NS, zdrain, 0)
            plsc.subcore_barrier()
            hop(e_h)
            plsc.subcore_barrier()

            dbufs = ((sbuf, lsem0, isem0), (tbuf, lsem1, isem1),
                     (gbuf0.at[pl.ds(0, _RB)], lsem2, gsem0),
                     (gbuf1.at[pl.ds(0, _RB)], lsem3, gsem1))

            def dump(dst_h):
                def dloop(g, _):
                    for bi, (buf, lsem, wsem) in enumerate(dbufs):
                        r0 = slab0 + (g * 4 + bi) * _RB

                        @pl.when(g > 0)
                        def _wait_store():
                            pltpu.make_async_copy(buf, dst_h.at[pl.ds(0, _RB)],
                                                  wsem).wait()

                        pltpu.async_copy(acc.at[pl.ds(r0, _RB)], buf, lsem)
                    for bi, (buf, lsem, wsem) in enumerate(dbufs):
                        r0 = slab0 + (g * 4 + bi) * _RB
                        pltpu.make_async_copy(acc.at[pl.ds(0, _RB)], buf,
                                              lsem).wait()
                        pltpu.async_copy(buf, dst_h.at[pl.ds(r0, _RB)], wsem)
                    return 0

                lax.fori_loop(0, _NS // 4, dloop, 0)
                for buf, lsem, wsem in dbufs:
                    pltpu.make_async_copy(buf, dst_h.at[pl.ds(0, _RB)],
                                          wsem).wait()

            dump(e1_h)
            plsc.subcore_barrier()
            hop(e1_h)
            plsc.subcore_barrier()
            dump(out_h)

        @pl.when(c == 0)
        def _core0():
            run(eA_h, e1A_h, outA_h)

        @pl.when(c == 1)
        def _core1():
            run(eB_h, e1B_h, outB_h)

    return spmm(eA, eB, rows, cols, vals)


def _spmm_chain(A_indices, A_values, e0):
    rpad = _NT_PAD - N_TOTAL
    e0p = jnp.pad(e0, ((0, rpad), (0, 0)))
    eA = e0p[:, :_HALF]
    eB = e0p[:, _HALF:]
    epad = _NNZ_PAD - NNZ
    rows = jnp.pad(A_indices[0].astype(jnp.int32), (0, epad))
    cols = jnp.pad(A_indices[1].astype(jnp.int32), (0, epad))
    vals = jnp.pad(A_values, (0, epad))
    outA, outB, _, _ = _spmm_sc_kernel(eA, eB, rows, cols, vals)
    w = jnp.concatenate([outA[:N_TOTAL], outB[:N_TOTAL]], axis=1)
    return (e0 + w) * (1.0 / 3.0)


def kernel(users, pos, neg, S, A_indices, A_values,
           user_embs_weight, item_embs_weight):
    u = user_embs_weight
    it = item_embs_weight

    e0 = jnp.concatenate([u, it], axis=0)
    final = _spmm_chain(A_indices, A_values, e0)
    fu_base = final[:N_USERS]
    fi = final[N_USERS:]

    h1, d = _graph_pass1(S, u)
    x2 = jnp.concatenate([h1, fu_base], axis=1)
    y2 = _graph_pass2(S, u, x2, d)
    h2 = y2[:, :HIDDEN]
    t2 = y2[:, HIDDEN:]

    view1 = (u + h1 + h2) * (1.0 / 3.0)
    fu = fu_base + t2

    u4 = jnp.concatenate([fu_base, view1, fu, u], axis=1)
    g = u4[users]
    pn = jnp.concatenate([pos, neg])
    fpn = fi[pn]
    ipn = it[pn]

    loss = _loss_kernel(g, fpn, ipn)
    return jnp.reshape(loss, ())


# speedup vs baseline: 1.0006x; 1.0006x over previous
import functools

import jax
import jax.numpy as jnp
from jax import lax
from jax.experimental import pallas as pl
from jax.experimental.pallas import tpu as pltpu

N_USERS = 5000
N_ITEMS = 50000
N_TOTAL = N_USERS + N_ITEMS
HIDDEN = 64
NNZ = 640000
BATCH = 4096
DECAY = 1e-4
CL_REG = 0.01
TEMP = 0.2


_R = 200
_CHUNKS = ((0, 1280), (1280, 1280), (2560, 1280), (3840, 1160))


def _normalize_rows(x):
    return x / (jnp.sqrt(jnp.sum(x * x, axis=1, keepdims=True)) + 1e-8)


def _pass1_body(s_ref, ui_ref, u_ref, x_ref, y_ref, d_ref):
    uni = _normalize_rows(ui_ref[...])
    acc = jnp.zeros((_R, HIDDEN), jnp.float32)
    accd = jnp.zeros((_R, 1), jnp.float32)
    for c0, w in _CHUNKS:
        unj = _normalize_rows(u_ref[c0:c0 + w, :])
        sim = jax.nn.relu(jax.lax.dot_general(
            uni, unj, (((1,), (1,)), ((), ())),
            preferred_element_type=jnp.float32))
        m = 0.5 * s_ref[:, c0:c0 + w] + 0.5 * sim
        acc += jnp.dot(m, x_ref[c0:c0 + w, :],
                       preferred_element_type=jnp.float32)
        accd += jnp.sum(m, axis=1, keepdims=True)
    d = accd + 1e-8
    y_ref[...] = acc / d
    d_ref[...] = d


def _pass2_body(s_ref, ui_ref, u_ref, x_ref, din_ref, y_ref):
    uni = _normalize_rows(ui_ref[...])
    k2 = x_ref.shape[1]
    acc = jnp.zeros((_R, k2), jnp.float32)
    for c0, w in _CHUNKS:
        unj = _normalize_rows(u_ref[c0:c0 + w, :])
        sim = jax.nn.relu(jax.lax.dot_general(
            uni, unj, (((1,), (1,)), ((), ())),
            preferred_element_type=jnp.float32))
        m = 0.5 * s_ref[:, c0:c0 + w] + 0.5 * sim
        acc += jnp.dot(m, x_ref[c0:c0 + w, :],
                       preferred_element_type=jnp.float32)
    y_ref[...] = acc / din_ref[...]


def _graph_pass1(S, u):
    ni = N_USERS // _R
    return pl.pallas_call(
        _pass1_body,
        grid=(ni,),
        in_specs=[
            pl.BlockSpec((_R, N_USERS), lambda i: (i, 0)),
            pl.BlockSpec((_R, HIDDEN), lambda i: (i, 0)),
            pl.BlockSpec((N_USERS, HIDDEN), lambda i: (0, 0)),
            pl.BlockSpec((N_USERS, HIDDEN), lambda i: (0, 0)),
        ],
        out_specs=[
            pl.BlockSpec((_R, HIDDEN), lambda i: (i, 0)),
            pl.BlockSpec((_R, 1), lambda i: (i, 0)),
        ],
        out_shape=[
            jax.ShapeDtypeStruct((N_USERS, HIDDEN), jnp.float32),
            jax.ShapeDtypeStruct((N_USERS, 1), jnp.float32),
        ],
    )(S, u, u, u)


def _graph_pass2(S, u, x2, d):
    ni = N_USERS // _R
    k2 = x2.shape[1]
    return pl.pallas_call(
        _pass2_body,
        grid=(ni,),
        in_specs=[
            pl.BlockSpec((_R, N_USERS), lambda i: (i, 0)),
            pl.BlockSpec((_R, HIDDEN), lambda i: (i, 0)),
            pl.BlockSpec((N_USERS, HIDDEN), lambda i: (0, 0)),
            pl.BlockSpec((N_USERS, k2), lambda i: (0, 0)),
            pl.BlockSpec((_R, 1), lambda i: (i, 0)),
        ],
        out_specs=pl.BlockSpec((_R, k2), lambda i: (i, 0)),
        out_shape=jax.ShapeDtypeStruct((N_USERS, k2), jnp.float32),
    )(S, u, u, x2, d)


_B = 512


def _loss_body(gi_ref, gj_ref, fp_ref, fn_ref, ip_ref, in_ref,
               out_ref, acct_ref, accs_ref):
    i = pl.program_id(0)
    j = pl.program_id(1)
    ni = pl.num_programs(0)
    nj = pl.num_programs(1)

    @pl.when(j == 0)
    def _init():
        acct_ref[...] = jnp.zeros_like(acct_ref)

    gi = gi_ref[...]
    z1n = _normalize_rows(gi[:, HIDDEN:2 * HIDDEN])
    z2n = _normalize_rows(gj_ref[:, 0:HIDDEN])
    logits = jax.lax.dot_general(
        z1n, z2n, (((1,), (1,)), ((), ())), preferred_element_type=jnp.float32)
    acct_ref[...] += jnp.sum(jnp.exp(logits / TEMP), axis=1, keepdims=True)

    @pl.when(j == nj - 1)
    def _fini():
        z2ni = _normalize_rows(gi[:, 0:HIDDEN])
        posv = jnp.exp(jnp.sum(z1n * z2ni, axis=1, keepdims=True) / TEMP)
        tot = acct_ref[...]
        cl_part = -jnp.sum(jnp.log(posv / (tot + 1e-8) + 1e-8)) / BATCH

        uo = gi[:, 2 * HIDDEN:3 * HIDDEN]
        ps = jnp.sum(uo * fp_ref[...], axis=1)
        ns = jnp.sum(uo * fn_ref[...], axis=1)
        x = ns - ps
        softplus = jnp.maximum(x, 0.0) + jnp.log1p(jnp.exp(-jnp.abs(x)))
        bpr_part = jnp.sum(softplus) / BATCH

        ue = gi[:, 3 * HIDDEN:4 * HIDDEN]
        reg_part = 0.5 * (jnp.sum(ue ** 2) + jnp.sum(ip_ref[...] ** 2)
                          + jnp.sum(in_ref[...] ** 2)) / BATCH

        partial = bpr_part + DECAY * reg_part + CL_REG * cl_part

        @pl.when(i == 0)
        def _zero():
            accs_ref[0] = 0.0

        accs_ref[0] += partial

        @pl.when(i == ni - 1)
        def _emit():
            out_ref[0, 0] = accs_ref[0]


def _loss_kernel(G, FPN, IPN):
    ni = nj = BATCH // _B
    nb = ni
    spec_gi = pl.BlockSpec((_B, 4 * HIDDEN), lambda i, j: (i, 0))
    spec_gj = pl.BlockSpec((_B, 4 * HIDDEN), lambda i, j: (j, 0))
    spec_p = pl.BlockSpec((_B, HIDDEN), lambda i, j: (i, 0))
    spec_n = pl.BlockSpec((_B, HIDDEN), lambda i, j: (i + nb, 0))
    out = pl.pallas_call(
        _loss_body,
        grid=(ni, nj),
        in_specs=[spec_gi, spec_gj, spec_p, spec_n, spec_p, spec_n],
        out_specs=pl.BlockSpec(memory_space=pltpu.SMEM),
        out_shape=jax.ShapeDtypeStruct((1, 1), jnp.float32),
        scratch_shapes=[
            pltpu.VMEM((_B, 1), jnp.float32),
            pltpu.SMEM((1,), jnp.float32),
        ],
    )(G, G, FPN, FPN, IPN, IPN)
    return out[0, 0]


from jax.experimental.pallas import tpu_sc as plsc

_NT_PAD = 55040
_RP = _NT_PAD // 16
_RB = 20
_EB = 256
_CH = 40960
_NB = _CH // _EB
_NNZ_PAD = 16 * _CH
_HALF = HIDDEN // 2


def _spmm_sc_kernel(eA, eB, rows, cols, vals):
    mesh = plsc.VectorSubcoreMesh(core_axis_name="c", subcore_axis_name="s")
    fdt = jnp.float32
    idt = jnp.int32
    out_t = [jax.ShapeDtypeStruct((_NT_PAD, _HALF), fdt) for _ in range(4)]

    @functools.partial(
        pl.kernel,
        out_type=out_t,
        mesh=mesh,
        compiler_params=pltpu.CompilerParams(use_tc_tiling_on_sc=False),
        scratch_types=[
            pltpu.VMEM_SHARED((_NT_PAD, _HALF), fdt),
            pltpu.VMEM((_EB,), idt), pltpu.VMEM((_EB,), idt),
            pltpu.VMEM((_EB,), idt), pltpu.VMEM((_EB,), idt),
            pltpu.VMEM((_EB,), fdt), pltpu.VMEM((_EB,), fdt),
            pltpu.VMEM((_EB,), idt), pltpu.VMEM((_EB,), idt),
            pltpu.VMEM((_EB, _HALF), fdt), pltpu.VMEM((_EB, _HALF), fdt),
            pltpu.VMEM((_RB, _HALF), fdt),
            pltpu.VMEM((_RB, _HALF), fdt),
            pltpu.SemaphoreType.DMA, pltpu.SemaphoreType.DMA,
            pltpu.SemaphoreType.DMA, pltpu.SemaphoreType.DMA,
            pltpu.SemaphoreType.DMA, pltpu.SemaphoreType.DMA,
            pltpu.SemaphoreType.DMA, pltpu.SemaphoreType.DMA,
            pltpu.SemaphoreType.DMA, pltpu.SemaphoreType.DMA,
        ],
    )
    def spmm(eA_h, eB_h, rows_h, cols_h, vals_h,
             outA_h, outB_h, e1A_h, e1B_h,
             acc, rowi0, rowi1, coli0, coli1, valb0, valb1, srow0, srow1,
             gbuf0, gbuf1, sbuf, tbuf,
             isem0, isem1, gsem0, gsem1, ssem0, ssem1,
             lsem0, lsem1, lsem2, lsem3):
        c = lax.axis_index("c")
        t = lax.axis_index("s")
        zeros16 = jnp.zeros((16,), fdt)

        def zero_slab(r, _):
            sbuf[r, 0:16] = zeros16
            sbuf[r, 16:32] = zeros16
            return 0

        bufs = ((rowi0, coli0, valb0, srow0, gbuf0, isem0, gsem0, ssem0),
                (rowi1, coli1, valb1, srow1, gbuf1, isem1, gsem1, ssem1))

        def fetch(b, ri, ci, vb, sem):
            base = t * _CH + b * _EB
            pltpu.async_copy(rows_h.at[pl.ds(base, _EB)], ri, sem)
            pltpu.async_copy(cols_h.at[pl.ds(base, _EB)], ci, sem)
            pltpu.async_copy(vals_h.at[pl.ds(base, _EB)], vb, sem)

        def fetch_wait(ri, ci, vb, sem):
            pltpu.make_async_copy(rows_h.at[pl.ds(0, _EB)], ri, sem).wait()
            pltpu.make_async_copy(cols_h.at[pl.ds(0, _EB)], ci, sem).wait()
            pltpu.make_async_copy(vals_h.at[pl.ds(0, _EB)], vb, sem).wait()

        def scale_copy(gb, vb, ri, sr):
            def scale(g, _):
                v16 = vb[pl.ds(g * 16, 16)]
                sr[pl.ds(g * 16, 16)] = ri[pl.ds(g * 16, 16)]
                for k in range(16):
                    vk = lax.gather(
                        v16, jnp.full((16, 1), k, jnp.int32),
                        lax.GatherDimensionNumbers(
                            offset_dims=(), collapsed_slice_dims=(0,),
                            start_index_map=(0,)),
                        slice_sizes=(1,),
                        mode=lax.GatherScatterMode.PROMISE_IN_BOUNDS)
                    e = g * 16 + k
                    gb[e, 0:16] *= vk
                    gb[e, 16:32] *= vk
                return 0

            lax.fori_loop(0, _EB // 16, scale, 0)

        def hop(src_h):
            def step(b, p):
                ri, ci, vb, sr, gb, isem, gsem, ssem = bufs[p]
                ri2, ci2, vb2, sr2, gb2, isem2, gsem2, ssem2 = bufs[1 - p]
                pltpu.make_async_copy(src_h.at[ci], gb, gsem).wait()

                @pl.when(b + 1 < _NB)
                def _issue_next():
                    fetch_wait(ri2, ci2, vb2, isem2)

                    @pl.when(b >= 1)
                    def _drain_prev_scatter():
                        pltpu.make_async_copy(gb2, acc.at[sr2], ssem2).wait()

                    pltpu.async_copy(src_h.at[ci2], gb2, gsem2)

                scale_copy(gb, vb, ri, sr)
                pltpu.async_copy(gb, acc.at[sr], ssem, add=True)

                @pl.when(b + 2 < _NB)
                def _prefetch():
                    fetch(b + 2, ri, ci, vb, isem)

            def pair(b2, _):
                step(2 * b2, 0)
                step(2 * b2 + 1, 1)
                return 0

            fetch(0, rowi0, coli0, valb0, isem0)
            fetch(1, rowi1, coli1, valb1, isem1)
            fetch_wait(rowi0, coli0, valb0, isem0)
            pltpu.async_copy(src_h.at[coli0], gbuf0, gsem0)
            lax.fori_loop(0, _NB // 2, pair, 0)
            pltpu.make_async_copy(gbuf0, acc.at[srow0], ssem0).wait()
            pltpu.make_async_copy(gbuf1, acc.at[srow1], ssem1).wait()

        _NS = _RP // _RB

        def run(e_h, e1_h, out_h):
            slab0 = t * _RP

            lax.fori_loop(0, _RB, zero_slab, 0)

            def zfire(s, _):
                pltpu.async_copy(sbuf, acc.at[pl.ds(slab0 + s * _RB, _RB)],
                                 ssem0)
                return 0

            def zdrain(s, _):
                pltpu.make_async_copy(sbuf, acc.at[pl.ds(0, _RB)],
                                      ssem0).wait()
                return 0

            lax.fori_loop(0, _NS, zfire, 0)
            lax.fori_loop(0, _NS, zdrain, 0)
            plsc.subcore_barrier()
            hop(e_h)
            plsc.subcore_barrier()

            dbufs = ((sbuf, lsem0, isem0), (tbuf, lsem1, isem1),
                     (gbuf0.at[pl.ds(0, _RB)], lsem2, gsem0),
                     (gbuf1.at[pl.ds(0, _RB)], lsem3, gsem1))

            def dump(dst_h):
                def dloop(g, _):
                    for bi, (buf, lsem, wsem) in enumerate(dbufs):
                        r0 = slab0 + (g * 4 + bi) * _RB

                        @pl.when(g > 0)
                        def _wait_store():
                            pltpu.make_async_copy(buf, dst_h.at[pl.ds(0, _RB)],
                                                  wsem).wait()

                        pltpu.async_copy(acc.at[pl.ds(r0, _RB)], buf, lsem)
                    for bi, (buf, lsem, wsem) in enumerate(dbufs):
                        r0 = slab0 + (g * 4 + bi) * _RB
                        pltpu.make_async_copy(acc.at[pl.ds(0, _RB)], buf,
                                              lsem).wait()
                        pltpu.async_copy(buf, dst_h.at[pl.ds(r0, _RB)], wsem)
                    return 0

                lax.fori_loop(0, _NS // 4, dloop, 0)
                for buf, lsem, wsem in dbufs:
                    pltpu.make_async_copy(buf, dst_h.at[pl.ds(0, _RB)],
                                          wsem).wait()

            dump(e1_h)
            plsc.subcore_barrier()
            hop(e1_h)
            plsc.subcore_barrier()
            dump(out_h)

        @pl.when(c == 0)
        def _core0():
            run(eA_h, e1A_h, outA_h)

        @pl.when(c == 1)
        def _core1():
            run(eB_h, e1B_h, outB_h)

    return spmm(eA, eB, rows, cols, vals)


def _spmm_chain(A_indices, A_values, e0):
    rpad = _NT_PAD - N_TOTAL
    e0p = jnp.pad(e0, ((0, rpad), (0, 0)))
    eA = e0p[:, :_HALF]
    eB = e0p[:, _HALF:]
    epad = _NNZ_PAD - NNZ
    rows = jnp.pad(A_indices[0].astype(jnp.int32), (0, epad))
    cols = jnp.pad(A_indices[1].astype(jnp.int32), (0, epad))
    vals = jnp.pad(A_values, (0, epad))
    outA, outB, _, _ = _spmm_sc_kernel(eA, eB, rows, cols, vals)
    w = jnp.concatenate([outA[:N_TOTAL], outB[:N_TOTAL]], axis=1)
    return (e0 + w) * (1.0 / 3.0)


def kernel(users, pos, neg, S, A_indices, A_values,
           user_embs_weight, item_embs_weight):
    u = user_embs_weight
    it = item_embs_weight

    e0 = jnp.concatenate([u, it], axis=0)
    final = _spmm_chain(A_indices, A_values, e0)
    fu_base = final[:N_USERS]
    fi = final[N_USERS:]

    h1, d = _graph_pass1(S, u)
    x2 = jnp.concatenate([h1, fu_base], axis=1)
    y2 = _graph_pass2(S, u, x2, d)
    h2 = y2[:, :HIDDEN]
    t2 = y2[:, HIDDEN:]

    view1 = (u + h1 + h2) * (1.0 / 3.0)
    fu = fu_base + t2

    u4 = jnp.concatenate([fu_base, view1, fu, u], axis=1)
    g = u4[users]
    pn = jnp.concatenate([pos, neg])
    fpn = fi[pn]
    ipn = it[pn]

    loss = _loss_kernel(g, fpn, ipn)
    return jnp.reshape(loss, ())
